# baseline (speedup 1.0000x reference)
import functools

import jax
import jax.numpy as jnp
from jax import lax
from jax.experimental import pallas as pl
from jax.experimental.pallas import tpu as pltpu
from jax.experimental.pallas import tpu_sc as plsc

N = 10000
E = 320000
D_IN = 128
H = 200
W = 128
NB = 2
NC = 2
NS = 16
L = 16
EPT = E // NS
B = 80
NBLK = EPT // B
F32 = jnp.float32

_ZBASE = 624


def _proj(wh, whp_o, es_o, ed_o, a_s, a_d):
    whp_o[0] = wh[:, :W]
    whp_o[1] = jnp.concatenate([wh[:, W:H], jnp.ones((N, 2 * W - H), F32)],
                               axis=1)
    es_o[...] = jnp.dot(wh, a_s[...], preferred_element_type=F32)
    ed_o[...] = jnp.dot(wh, a_d[...], preferred_element_type=F32)


def _dense_in_body(feat, w_in, b_in, w_g, a_s, a_d, h0_o, whp_o, es_o, ed_o):
    h = jnp.dot(feat[...], w_in[...], preferred_element_type=F32) + b_in[...]
    h0_o[...] = h
    wh = jnp.dot(h, w_g[...], preferred_element_type=F32)
    _proj(wh, whp_o, es_o, ed_o, a_s, a_d)


def _norm(acc, b_g):
    agg = jnp.concatenate([acc[0], acc[1][:, :H - W]], axis=1)
    denom = acc[1][:, H - W:H - W + 1]
    return agg / (denom + 1e-16) + b_g[...]


def _dense_mid_body(acc, w_g, b_g, a_s, a_d, h1_o, whp_o, es_o, ed_o):
    h = _norm(acc, b_g)
    h1_o[...] = h
    wh = jnp.dot(h, w_g[...], preferred_element_type=F32)
    _proj(wh, whp_o, es_o, ed_o, a_s, a_d)


def _dense_out_body(acc, b_g, h0, h1, w_out, b_out, out_o):
    h2 = _norm(acc, b_g)
    out_o[...] = (
        jnp.dot(h0[...], w_out[0:H, :], preferred_element_type=F32)
        + jnp.dot(h1[...], w_out[H:2 * H, :], preferred_element_type=F32)
        + jnp.dot(h2, w_out[2 * H:3 * H, :], preferred_element_type=F32)
        + b_out[...]
    )


def _dense_in(feat, w_in, b_in, w_g, a_s, a_d):
    return pl.pallas_call(
        _dense_in_body,
        out_shape=[
            jax.ShapeDtypeStruct((N, H), F32),
            jax.ShapeDtypeStruct((NB, N, W), F32),
            jax.ShapeDtypeStruct((N, 1), F32),
            jax.ShapeDtypeStruct((N, 1), F32),
        ],
    )(feat, w_in, b_in, w_g, a_s, a_d)


def _dense_mid(acc, w_g, b_g, a_s, a_d):
    return pl.pallas_call(
        _dense_mid_body,
        out_shape=[
            jax.ShapeDtypeStruct((N, H), F32),
            jax.ShapeDtypeStruct((NB, N, W), F32),
            jax.ShapeDtypeStruct((N, 1), F32),
            jax.ShapeDtypeStruct((N, 1), F32),
        ],
    )(acc, w_g, b_g, a_s, a_d)


def _dense_out(acc, b_g, h0, h1, w_out, b_out):
    return pl.pallas_call(
        _dense_out_body,
        out_shape=jax.ShapeDtypeStruct((N, H), F32),
    )(acc, b_g, h0, h1, w_out, b_out)


TILES = NC * NS
BPT_W = E // TILES // B
BA = 125
PB = 128
NBLKA = (E // NS) // BA
ACCR = N + 8


def _sc_w_body(src_hbm, dst_hbm, es_hbm, ed_hbm, w_hbm,
               es_v, ed_v, src_a, dst_a, w_a):
    c = lax.axis_index("c")
    s = lax.axis_index("s")
    tid = c * NS + s

    pltpu.sync_copy(es_hbm, es_v)
    pltpu.sync_copy(ed_hbm, ed_v)
    pltpu.sync_copy(src_hbm.at[tid], src_a)
    pltpu.sync_copy(dst_hbm.at[tid], dst_a)

    def _wblk(b, carry):
        for g in range(B // L):
            sl = pl.ds(g * L, L)
            si = src_a[b, sl]
            di = dst_a[b, sl]
            x = plsc.load_gather(es_v, [si]) + plsc.load_gather(ed_v, [di])
            xl = jnp.where(x > 0, x, 0.2 * x)
            w_a[b, sl] = jnp.exp(xl)
        return carry
    lax.fori_loop(0, BPT_W, _wblk, 0)

    pltpu.sync_copy(w_a, w_hbm.at[tid])


@functools.cache
def _sc_w():
    mesh = plsc.VectorSubcoreMesh(
        core_axis_name="c", subcore_axis_name="s",
        num_cores=NC, num_subcores=NS)
    return pl.kernel(
        _sc_w_body,
        out_type=jax.ShapeDtypeStruct((TILES, BPT_W, B), F32),
        mesh=mesh,
        compiler_params=pltpu.CompilerParams(
            needs_layout_passes=False, use_tc_tiling_on_sc=False),
        scratch_types=[
            pltpu.VMEM((N,), F32),
            pltpu.VMEM((N,), F32),
            pltpu.VMEM((BPT_W, B), jnp.int32),
            pltpu.VMEM((BPT_W, B), jnp.int32),
            pltpu.VMEM((BPT_W, B), F32),
        ],
    )


def _sc_agg_body(idx_hbm, whp_hbm, out_hbm,
                 sdw0, sdw1, sdw2, dsc0, dsc1, dsc2, g0, g1, g2, acc,
                 is0, is1, is2, gs0, gs1, gs2, ss0, ss1, ss2):
    c = lax.axis_index("c")
    s = lax.axis_index("s")
    sdw = (sdw0, sdw1, sdw2)
    gb = (g0, g1, g2)
    dsc = (dsc0, dsc1, dsc2)
    isem = (is0, is1, is2)
    gsem = (gs0, gs1, gs2)
    ssem = (ss0, ss1, ss2)

    def _zero_row(i, carry):
        for cc in range(W // L):
            g0[i, pl.ds(cc * L, L)] = jnp.zeros((L,), F32)
        return carry
    lax.fori_loop(0, PB, _zero_row, 0)

    rbase = s * _ZBASE
    nfull = jnp.where(s == NS - 1, 5, 4)
    rem = jnp.where(s == NS - 1, 0, _ZBASE - 4 * PB)

    def _zacc(k, carry):
        pltpu.sync_copy(g0, acc.at[pl.ds(rbase + k * PB, PB)])
        return carry
    lax.fori_loop(0, nfull, _zacc, 0)

    @pl.when(rem > 0)
    def _():
        pltpu.sync_copy(g0.at[pl.ds(0, _ZBASE - 4 * PB)],
                        acc.at[pl.ds(rbase + 4 * PB, _ZBASE - 4 * PB)])

    plsc.subcore_barrier()

    bbase = s * NBLKA
    rowoff = c * N

    def _rebase_and_gather(p, gsl):
        for g in range(PB // L):
            sl = pl.ds(g * L, L)
            sdw[p][sl] = sdw[p][sl] + rowoff
        pltpu.async_copy(whp_hbm.at[sdw[p].at[pl.ds(0, PB)]], gb[gsl],
                         gsem[gsl])

    for p in range(3):
        pltpu.async_copy(idx_hbm.at[bbase + p], sdw[p], isem[p])
    for p in range(2):
        pltpu.make_async_copy(idx_hbm.at[bbase], sdw[p], isem[p]).wait()
        _rebase_and_gather(p, p)

    def _halfstep(b, p):
        pltpu.make_async_copy(whp_hbm.at[sdw[p].at[pl.ds(0, PB)]],
                              gb[p], gsem[p]).wait()
        for g in range(PB // L):
            sl = pl.ds(g * L, L)
            dsc[p][sl] = sdw[p][pl.ds(PB + g * L, L)]
        @plsc.parallel_loop(0, PB, 1, unroll=4)
        def _(e2):
            wi = plsc.load_gather(
                sdw[p], [jnp.full((L,), 2 * PB + e2, jnp.int32)])
            wb = plsc.bitcast(wi, F32)
            for cc in range(W // L):
                csl = pl.ds(cc * L, L)
                gb[p][e2, csl] = gb[p][e2, csl] * wb
        pltpu.async_copy(gb[p], acc.at[dsc[p]], ssem[p], add=True)
        @pl.when(b + 2 < NBLKA)
        def _():
            p2 = (p + 2) % 3
            pltpu.make_async_copy(idx_hbm.at[bbase], sdw[p2],
                                  isem[p2]).wait()
            @pl.when(b >= 1)
            def _():
                pltpu.make_async_copy(gb[p2], acc.at[dsc[p2]],
                                      ssem[p2]).wait()
            _rebase_and_gather(p2, p2)
        @pl.when(b + 3 < NBLKA)
        def _():
            pltpu.async_copy(idx_hbm.at[bbase + b + 3], sdw[p], isem[p])

    def _triple(i, carry):
        b0 = 3 * i
        _halfstep(b0, 0)
        _halfstep(b0 + 1, 1)
        _halfstep(b0 + 2, 2)
        return carry
    lax.fori_loop(0, (NBLKA - 1) // 3, _triple, 0)
    _halfstep(NBLKA - 1, (NBLKA - 1) % 3)

    for p in (1, 2, 0):
        pltpu.make_async_copy(gb[p], acc.at[dsc[p]], ssem[p]).wait()

    plsc.subcore_barrier()

    obase = c * N + rbase

    def _wout(k, carry):
        pltpu.sync_copy(acc.at[pl.ds(rbase + k * PB, PB)],
                        out_hbm.at[pl.ds(obase + k * PB, PB)])
        return carry
    lax.fori_loop(0, nfull, _wout, 0)

    @pl.when(rem > 0)
    def _():
        pltpu.sync_copy(acc.at[pl.ds(rbase + 4 * PB, _ZBASE - 4 * PB)],
                        out_hbm.at[pl.ds(obase + 4 * PB, _ZBASE - 4 * PB)])


@functools.cache
def _sc_agg():
    mesh = plsc.VectorSubcoreMesh(
        core_axis_name="c", subcore_axis_name="s",
        num_cores=NC, num_subcores=NS)
    return pl.kernel(
        _sc_agg_body,
        out_type=jax.ShapeDtypeStruct((NC * N, W), F32),
        mesh=mesh,
        compiler_params=pltpu.CompilerParams(
            needs_layout_passes=False, use_tc_tiling_on_sc=False),
        scratch_types=(
            [pltpu.VMEM((3 * PB,), jnp.int32) for _ in range(3)]
            + [pltpu.VMEM((PB,), jnp.int32) for _ in range(3)]
            + [pltpu.VMEM((PB, W), F32) for _ in range(3)]
            + [pltpu.VMEM_SHARED((N, W), F32)]
            + [pltpu.SemaphoreType.DMA for _ in range(9)]
        ),
    )


def kernel(features, edge_index, W_in, b_in, W_g, a_src, a_dst, b_g,
           W_out, b_out):
    src3 = edge_index[0].astype(jnp.int32).reshape(TILES, BPT_W, B)
    dst3 = edge_index[1].astype(jnp.int32).reshape(TILES, BPT_W, B)
    srcp = jnp.pad(src3.reshape(E // BA, BA), ((0, 0), (0, PB - BA)))
    dstp = jnp.pad(dst3.reshape(E // BA, BA), ((0, 0), (0, PB - BA)))

    def _pack_idx(w):
        wp = jnp.pad(w.reshape(E // BA, BA), ((0, 0), (0, PB - BA)))
        wi = jax.lax.bitcast_convert_type(wp, jnp.int32)
        return jnp.concatenate([srcp, dstp, wi], axis=1)
    b_in2 = b_in.reshape(1, H)
    b_g2 = b_g.reshape(1, H)
    b_out2 = b_out.reshape(1, H)
    a_s = a_src.reshape(H, 1)
    a_d = a_dst.reshape(H, 1)

    sc_w = _sc_w()
    sc_agg = _sc_agg()
    h0, whp0, es0, ed0 = _dense_in(features, W_in, b_in2, W_g, a_s, a_d)
    w0 = sc_w(src3, dst3, es0.reshape(N), ed0.reshape(N))
    acc0 = sc_agg(_pack_idx(w0), whp0.reshape(NB * N, W))
    h1, whp1, es1, ed1 = _dense_mid(acc0.reshape(NC, N, W), W_g, b_g2,
                                    a_s, a_d)
    w1 = sc_w(src3, dst3, es1.reshape(N), ed1.reshape(N))
    acc1 = sc_agg(_pack_idx(w1), whp1.reshape(NB * N, W))
    return _dense_out(acc1.reshape(NC, N, W), b_g2, h0, h1, W_out, b_out2)


# speedup vs baseline: 1.4795x; 1.4795x over previous
import functools

import jax
import jax.numpy as jnp
from jax import lax
from jax.experimental import pallas as pl
from jax.experimental.pallas import tpu as pltpu
from jax.experimental.pallas import tpu_sc as plsc

N = 10000
E = 320000
D_IN = 128
H = 200
W = 128
NB = 2
NC = 2
NS = 16
L = 16
EPT = E // NS
B = 80
NBLK = EPT // B
F32 = jnp.float32

_ZBASE = 624


def _proj(wh, whp_o, es_o, ed_o, a_s, a_d):
    whp_o[0] = wh[:, :W]
    whp_o[1] = jnp.concatenate([wh[:, W:H], jnp.ones((N, 2 * W - H), F32)],
                               axis=1)
    es_o[...] = jnp.dot(wh, a_s[...], preferred_element_type=F32)
    ed_o[...] = jnp.dot(wh, a_d[...], preferred_element_type=F32)


def _dense_in_body(feat, w_in, b_in, w_g, a_s, a_d, h0_o, whp_o, es_o, ed_o):
    h = jnp.dot(feat[...], w_in[...], preferred_element_type=F32) + b_in[...]
    h0_o[...] = h
    wh = jnp.dot(h, w_g[...], preferred_element_type=F32)
    _proj(wh, whp_o, es_o, ed_o, a_s, a_d)


def _norm(acc, b_g):
    agg = jnp.concatenate([acc[0], acc[1][:, :H - W]], axis=1)
    denom = acc[1][:, H - W:H - W + 1]
    return agg / (denom + 1e-16) + b_g[...]


def _dense_mid_body(acc, w_g, b_g, a_s, a_d, h1_o, whp_o, es_o, ed_o):
    h = _norm(acc, b_g)
    h1_o[...] = h
    wh = jnp.dot(h, w_g[...], preferred_element_type=F32)
    _proj(wh, whp_o, es_o, ed_o, a_s, a_d)


def _dense_out_body(acc, b_g, h0, h1, w_out, b_out, out_o):
    h2 = _norm(acc, b_g)
    out_o[...] = (
        jnp.dot(h0[...], w_out[0:H, :], preferred_element_type=F32)
        + jnp.dot(h1[...], w_out[H:2 * H, :], preferred_element_type=F32)
        + jnp.dot(h2, w_out[2 * H:3 * H, :], preferred_element_type=F32)
        + b_out[...]
    )


def _dense_in(feat, w_in, b_in, w_g, a_s, a_d):
    return pl.pallas_call(
        _dense_in_body,
        out_shape=[
            jax.ShapeDtypeStruct((N, H), F32),
            jax.ShapeDtypeStruct((NB, N, W), F32),
            jax.ShapeDtypeStruct((N, 1), F32),
            jax.ShapeDtypeStruct((N, 1), F32),
        ],
    )(feat, w_in, b_in, w_g, a_s, a_d)


def _dense_mid(acc, w_g, b_g, a_s, a_d):
    return pl.pallas_call(
        _dense_mid_body,
        out_shape=[
            jax.ShapeDtypeStruct((N, H), F32),
            jax.ShapeDtypeStruct((NB, N, W), F32),
            jax.ShapeDtypeStruct((N, 1), F32),
            jax.ShapeDtypeStruct((N, 1), F32),
        ],
    )(acc, w_g, b_g, a_s, a_d)


def _dense_out(acc, b_g, h0, h1, w_out, b_out):
    return pl.pallas_call(
        _dense_out_body,
        out_shape=jax.ShapeDtypeStruct((N, H), F32),
    )(acc, b_g, h0, h1, w_out, b_out)


TILES = NC * NS
BPT_W = E // TILES // B
BA = 125
PB = 128
NBLKA = (E // NS) // BA
ACCR = N + 8


def _sc_w_body(src_hbm, dst_hbm, es_hbm, ed_hbm, w_hbm,
               es_v, ed_v, src_a, dst_a, w_a):
    c = lax.axis_index("c")
    s = lax.axis_index("s")
    tid = c * NS + s

    pltpu.sync_copy(es_hbm, es_v)
    pltpu.sync_copy(ed_hbm, ed_v)
    pltpu.sync_copy(src_hbm.at[tid], src_a)
    pltpu.sync_copy(dst_hbm.at[tid], dst_a)

    def _wblk(b, carry):
        for g in range(B // L):
            sl = pl.ds(g * L, L)
            si = src_a[b, sl]
            di = dst_a[b, sl]
            x = plsc.load_gather(es_v, [si]) + plsc.load_gather(ed_v, [di])
            xl = jnp.where(x > 0, x, 0.2 * x)
            w_a[b, sl] = jnp.exp(xl)
        return carry
    lax.fori_loop(0, BPT_W, _wblk, 0)

    pltpu.sync_copy(w_a, w_hbm.at[tid])


@functools.cache
def _sc_w():
    mesh = plsc.VectorSubcoreMesh(
        core_axis_name="c", subcore_axis_name="s",
        num_cores=NC, num_subcores=NS)
    return pl.kernel(
        _sc_w_body,
        out_type=jax.ShapeDtypeStruct((TILES, BPT_W, B), F32),
        mesh=mesh,
        compiler_params=pltpu.CompilerParams(
            needs_layout_passes=False, use_tc_tiling_on_sc=False),
        scratch_types=[
            pltpu.VMEM((N,), F32),
            pltpu.VMEM((N,), F32),
            pltpu.VMEM((BPT_W, B), jnp.int32),
            pltpu.VMEM((BPT_W, B), jnp.int32),
            pltpu.VMEM((BPT_W, B), F32),
        ],
    )


def _sc_agg_body(src_hbm, dst_hbm, w_hbm, whp_hbm, out_hbm,
                 si0, si1, di0, di1, wv0, wv1, dsc0, dsc1,
                 g0, g1, s0, s1, acc,
                 is0, is1, gs0, gs1, ss0, ss1):
    c = lax.axis_index("c")
    s = lax.axis_index("s")

    def _zero_row(i, carry):
        for cc in range(W // L):
            s0[i, pl.ds(cc * L, L)] = jnp.zeros((L,), F32)
        return carry
    lax.fori_loop(0, B, _zero_row, 0)

    rbase = s * _ZBASE
    nrows = jnp.where(s == NS - 1, N - (NS - 1) * _ZBASE, _ZBASE)
    nfull = nrows // B
    rem = nrows - nfull * B

    def _zacc(k, carry):
        pltpu.sync_copy(s0, acc.at[pl.ds(rbase + k * B, B)])
        return carry
    lax.fori_loop(0, nfull, _zacc, 0)

    @pl.when(rem > 0)
    def _():
        pltpu.sync_copy(s0.at[pl.ds(0, 64)],
                        acc.at[pl.ds(rbase + nfull * B, 64)])

    plsc.subcore_barrier()

    bbase = s * NBLK
    rowoff = c * N

    def _issue_idx(b, sI, dI, wv, isem):
        pltpu.async_copy(src_hbm.at[bbase + b], sI, isem)
        pltpu.async_copy(dst_hbm.at[bbase + b], dI, isem)
        pltpu.async_copy(w_hbm.at[bbase + b], wv, isem)

    def _wait_idx(b, sI, dI, wv, isem):
        pltpu.make_async_copy(src_hbm.at[bbase + b], sI, isem).wait()
        pltpu.make_async_copy(dst_hbm.at[bbase + b], dI, isem).wait()
        pltpu.make_async_copy(w_hbm.at[bbase + b], wv, isem).wait()

    def _rebase_and_gather(sI, gbuf, gsem):
        for g in range(B // L):
            sl = pl.ds(g * L, L)
            sI[sl] = sI[sl] + rowoff
        pltpu.async_copy(whp_hbm.at[sI], gbuf, gsem)

    _issue_idx(0, si0, di0, wv0, is0)
    _issue_idx(1, si1, di1, wv1, is1)
    _wait_idx(0, si0, di0, wv0, is0)
    _rebase_and_gather(si0, g0, gs0)

    def _halfstep(b, sI, dI, wv, dsc, gbuf, sbuf, isem, gsem, ssem,
                  sIn, gn, gsn, isn):
        @pl.when(b + 1 < NBLK)
        def _():
            _wait_idx(b + 1, sIn, dsc, wv, isn)
            _rebase_and_gather(sIn, gn, gsn)
        pltpu.make_async_copy(whp_hbm.at[sI], gbuf, gsem).wait()
        @pl.when(b >= 2)
        def _():
            pltpu.make_async_copy(sbuf, acc.at[dsc], ssem).wait()
        @plsc.parallel_loop(0, B, 1, unroll=4)
        def _(e2):
            wb = plsc.load_gather(wv, [jnp.full((L,), e2, jnp.int32)])
            for cc in range(W // L):
                csl = pl.ds(cc * L, L)
                sbuf[e2, csl] = gbuf[e2, csl] * wb
        for g in range(B // L):
            sl = pl.ds(g * L, L)
            dsc[sl] = dI[sl]
        pltpu.async_copy(sbuf, acc.at[dsc], ssem, add=True)
        @pl.when(b + 2 < NBLK)
        def _():
            _issue_idx(b + 2, sI, dI, wv, isem)

    def _pair(i, carry):
        b0 = 2 * i
        _halfstep(b0, si0, di0, wv0, dsc0, g0, s0, is0, gs0, ss0,
                  si1, g1, gs1, is1)
        _halfstep(b0 + 1, si1, di1, wv1, dsc1, g1, s1, is1, gs1, ss1,
                  si0, g0, gs0, is0)
        return carry
    lax.fori_loop(0, NBLK // 2, _pair, 0)

    pltpu.make_async_copy(s0, acc.at[dsc0], ss0).wait()
    pltpu.make_async_copy(s1, acc.at[dsc1], ss1).wait()

    plsc.subcore_barrier()

    obase = c * N + rbase

    def _wout(k, carry):
        pltpu.sync_copy(acc.at[pl.ds(rbase + k * B, B)],
                        out_hbm.at[pl.ds(obase + k * B, B)])
        return carry
    lax.fori_loop(0, nfull, _wout, 0)

    @pl.when(rem > 0)
    def _():
        pltpu.sync_copy(acc.at[pl.ds(rbase + nfull * B, 64)],
                        out_hbm.at[pl.ds(obase + nfull * B, 64)])


@functools.cache
def _sc_agg():
    mesh = plsc.VectorSubcoreMesh(
        core_axis_name="c", subcore_axis_name="s",
        num_cores=NC, num_subcores=NS)
    return pl.kernel(
        _sc_agg_body,
        out_type=jax.ShapeDtypeStruct((NC * N, W), F32),
        mesh=mesh,
        compiler_params=pltpu.CompilerParams(
            needs_layout_passes=False, use_tc_tiling_on_sc=False),
        scratch_types=[
            pltpu.VMEM((B,), jnp.int32),
            pltpu.VMEM((B,), jnp.int32),
            pltpu.VMEM((B,), jnp.int32),
            pltpu.VMEM((B,), jnp.int32),
            pltpu.VMEM((B,), F32),
            pltpu.VMEM((B,), F32),
            pltpu.VMEM((B,), jnp.int32),
            pltpu.VMEM((B,), jnp.int32),
            pltpu.VMEM((B, W), F32),
            pltpu.VMEM((B, W), F32),
            pltpu.VMEM((B, W), F32),
            pltpu.VMEM((B, W), F32),
            pltpu.VMEM_SHARED((N, W), F32),
            pltpu.SemaphoreType.DMA,
            pltpu.SemaphoreType.DMA,
            pltpu.SemaphoreType.DMA,
            pltpu.SemaphoreType.DMA,
            pltpu.SemaphoreType.DMA,
            pltpu.SemaphoreType.DMA,
        ],
    )


def kernel(features, edge_index, W_in, b_in, W_g, a_src, a_dst, b_g,
           W_out, b_out):
    src2 = edge_index[0].astype(jnp.int32).reshape(E // B, B)
    dst2 = edge_index[1].astype(jnp.int32).reshape(E // B, B)
    src3 = src2.reshape(TILES, BPT_W, B)
    dst3 = dst2.reshape(TILES, BPT_W, B)
    b_in2 = b_in.reshape(1, H)
    b_g2 = b_g.reshape(1, H)
    b_out2 = b_out.reshape(1, H)
    a_s = a_src.reshape(H, 1)
    a_d = a_dst.reshape(H, 1)

    sc_w = _sc_w()
    sc_agg = _sc_agg()
    h0, whp0, es0, ed0 = _dense_in(features, W_in, b_in2, W_g, a_s, a_d)
    w0 = sc_w(src3, dst3, es0.reshape(N), ed0.reshape(N))
    acc0 = sc_agg(src2, dst2, w0.reshape(E // B, B),
                  whp0.reshape(NB * N, W))
    h1, whp1, es1, ed1 = _dense_mid(acc0.reshape(NC, N, W), W_g, b_g2,
                                    a_s, a_d)
    w1 = sc_w(src3, dst3, es1.reshape(N), ed1.reshape(N))
    acc1 = sc_agg(src2, dst2, w1.reshape(E // B, B),
                  whp1.reshape(NB * N, W))
    return _dense_out(acc1.reshape(NC, N, W), b_g2, h0, h1, W_out, b_out2)


# speedup vs baseline: 1.4810x; 1.0010x over previous
import functools

import jax
import jax.numpy as jnp
from jax import lax
from jax.experimental import pallas as pl
from jax.experimental.pallas import tpu as pltpu
from jax.experimental.pallas import tpu_sc as plsc

N = 10000
E = 320000
D_IN = 128
H = 200
W = 128
NB = 2
NC = 2
NS = 16
L = 16
EPT = E // NS
B = 80
NBLK = EPT // B
F32 = jnp.float32

_ZBASE = 624


def _proj(wh, whp_o, es_o, ed_o, a_s, a_d):
    whp_o[0] = wh[:, :W]
    whp_o[1] = jnp.concatenate([wh[:, W:H], jnp.ones((N, 2 * W - H), F32)],
                               axis=1)
    es_o[...] = jnp.dot(wh, a_s[...], preferred_element_type=F32)
    ed_o[...] = jnp.dot(wh, a_d[...], preferred_element_type=F32)


def _dense_in_body(feat, w_in, b_in, w_g, a_s, a_d, h0_o, whp_o, es_o, ed_o):
    h = jnp.dot(feat[...], w_in[...], preferred_element_type=F32) + b_in[...]
    h0_o[...] = h
    wh = jnp.dot(h, w_g[...], preferred_element_type=F32)
    _proj(wh, whp_o, es_o, ed_o, a_s, a_d)


def _norm(acc, b_g):
    agg = jnp.concatenate([acc[0], acc[1][:, :H - W]], axis=1)
    denom = acc[1][:, H - W:H - W + 1]
    return agg / (denom + 1e-16) + b_g[...]


def _dense_mid_body(acc, w_g, b_g, a_s, a_d, h1_o, whp_o, es_o, ed_o):
    h = _norm(acc, b_g)
    h1_o[...] = h
    wh = jnp.dot(h, w_g[...], preferred_element_type=F32)
    _proj(wh, whp_o, es_o, ed_o, a_s, a_d)


def _dense_out_body(acc, b_g, h0, h1, w_out, b_out, out_o):
    h2 = _norm(acc, b_g)
    out_o[...] = (
        jnp.dot(h0[...], w_out[0:H, :], preferred_element_type=F32)
        + jnp.dot(h1[...], w_out[H:2 * H, :], preferred_element_type=F32)
        + jnp.dot(h2, w_out[2 * H:3 * H, :], preferred_element_type=F32)
        + b_out[...]
    )


def _dense_in(feat, w_in, b_in, w_g, a_s, a_d):
    return pl.pallas_call(
        _dense_in_body,
        out_shape=[
            jax.ShapeDtypeStruct((N, H), F32),
            jax.ShapeDtypeStruct((NB, N, W), F32),
            jax.ShapeDtypeStruct((N, 1), F32),
            jax.ShapeDtypeStruct((N, 1), F32),
        ],
    )(feat, w_in, b_in, w_g, a_s, a_d)


def _dense_mid(acc, w_g, b_g, a_s, a_d):
    return pl.pallas_call(
        _dense_mid_body,
        out_shape=[
            jax.ShapeDtypeStruct((N, H), F32),
            jax.ShapeDtypeStruct((NB, N, W), F32),
            jax.ShapeDtypeStruct((N, 1), F32),
            jax.ShapeDtypeStruct((N, 1), F32),
        ],
    )(acc, w_g, b_g, a_s, a_d)


def _dense_out(acc, b_g, h0, h1, w_out, b_out):
    return pl.pallas_call(
        _dense_out_body,
        out_shape=jax.ShapeDtypeStruct((N, H), F32),
    )(acc, b_g, h0, h1, w_out, b_out)


TILES = NC * NS
BPT_W = E // TILES // B
BA = 125
PB = 128
NBLKA = (E // NS) // BA
ACCR = N + 8


def _sc_w_body(src_hbm, dst_hbm, es_hbm, ed_hbm, w_hbm,
               es_v, ed_v, src_a, dst_a, w_a):
    c = lax.axis_index("c")
    s = lax.axis_index("s")
    tid = c * NS + s

    pltpu.sync_copy(es_hbm, es_v)
    pltpu.sync_copy(ed_hbm, ed_v)
    pltpu.sync_copy(src_hbm.at[tid], src_a)
    pltpu.sync_copy(dst_hbm.at[tid], dst_a)

    def _wblk(b, carry):
        for g in range(B // L):
            sl = pl.ds(g * L, L)
            si = src_a[b, sl]
            di = dst_a[b, sl]
            x = plsc.load_gather(es_v, [si]) + plsc.load_gather(ed_v, [di])
            xl = jnp.where(x > 0, x, 0.2 * x)
            w_a[b, sl] = jnp.exp(xl)
        return carry
    lax.fori_loop(0, BPT_W, _wblk, 0)

    pltpu.sync_copy(w_a, w_hbm.at[tid])


@functools.cache
def _sc_w():
    mesh = plsc.VectorSubcoreMesh(
        core_axis_name="c", subcore_axis_name="s",
        num_cores=NC, num_subcores=NS)
    return pl.kernel(
        _sc_w_body,
        out_type=jax.ShapeDtypeStruct((TILES, BPT_W, B), F32),
        mesh=mesh,
        compiler_params=pltpu.CompilerParams(
            needs_layout_passes=False, use_tc_tiling_on_sc=False),
        scratch_types=[
            pltpu.VMEM((N,), F32),
            pltpu.VMEM((N,), F32),
            pltpu.VMEM((BPT_W, B), jnp.int32),
            pltpu.VMEM((BPT_W, B), jnp.int32),
            pltpu.VMEM((BPT_W, B), F32),
        ],
    )


def _sc_agg_body(src_hbm, dst_hbm, w_hbm, whp_hbm, out_hbm,
                 si0, si1, di0, di1, wv0, wv1, dsc0, dsc1,
                 g0, g1, s0, s1, acc,
                 is0, is1, gs0, gs1, ss0, ss1):
    c = lax.axis_index("c")
    s = lax.axis_index("s")

    def _zero_row(i, carry):
        for cc in range(W // L):
            s0[i, pl.ds(cc * L, L)] = jnp.zeros((L,), F32)
        return carry
    lax.fori_loop(0, B, _zero_row, 0)

    rbase = s * _ZBASE
    nrows = jnp.where(s == NS - 1, N - (NS - 1) * _ZBASE, _ZBASE)
    nfull = nrows // B
    rem = nrows - nfull * B

    def _zacc(k, carry):
        pltpu.sync_copy(s0, acc.at[pl.ds(rbase + k * B, B)])
        return carry
    lax.fori_loop(0, nfull, _zacc, 0)

    @pl.when(rem > 0)
    def _():
        pltpu.sync_copy(s0.at[pl.ds(0, 64)],
                        acc.at[pl.ds(rbase + nfull * B, 64)])

    plsc.subcore_barrier()

    bbase = s * NBLK
    rowoff = c * N

    def _issue_idx(b, sI, dI, wv, isem):
        pltpu.async_copy(src_hbm.at[bbase + b], sI, isem)
        pltpu.async_copy(dst_hbm.at[bbase + b], dI, isem)
        pltpu.async_copy(w_hbm.at[bbase + b], wv, isem)

    def _wait_idx(b, sI, dI, wv, isem):
        pltpu.make_async_copy(src_hbm.at[bbase + b], sI, isem).wait()
        pltpu.make_async_copy(dst_hbm.at[bbase + b], dI, isem).wait()
        pltpu.make_async_copy(w_hbm.at[bbase + b], wv, isem).wait()

    def _rebase_and_gather(sI, gbuf, gsem):
        for g in range(B // L):
            sl = pl.ds(g * L, L)
            sI[sl] = sI[sl] + rowoff
        pltpu.async_copy(whp_hbm.at[sI], gbuf, gsem)

    _issue_idx(0, si0, di0, wv0, is0)
    _issue_idx(1, si1, di1, wv1, is1)
    _wait_idx(0, si0, di0, wv0, is0)
    _rebase_and_gather(si0, g0, gs0)

    def _halfstep(b, sI, dI, wv, dsc, gbuf, sbuf, isem, gsem, ssem,
                  sIn, gn, gsn, isn):
        @pl.when(b + 1 < NBLK)
        def _():
            _wait_idx(b + 1, sIn, dsc, wv, isn)
            _rebase_and_gather(sIn, gn, gsn)
        pltpu.make_async_copy(whp_hbm.at[sI], gbuf, gsem).wait()
        @pl.when(b >= 2)
        def _():
            pltpu.make_async_copy(sbuf, acc.at[dsc], ssem).wait()
        @plsc.parallel_loop(0, B, 1, unroll=8)
        def _(e2):
            wb = plsc.load_gather(wv, [jnp.full((L,), e2, jnp.int32)])
            for cc in range(W // L):
                csl = pl.ds(cc * L, L)
                sbuf[e2, csl] = gbuf[e2, csl] * wb
        for g in range(B // L):
            sl = pl.ds(g * L, L)
            dsc[sl] = dI[sl]
        pltpu.async_copy(sbuf, acc.at[dsc], ssem, add=True)
        @pl.when(b + 2 < NBLK)
        def _():
            _issue_idx(b + 2, sI, dI, wv, isem)

    def _pair(i, carry):
        b0 = 2 * i
        _halfstep(b0, si0, di0, wv0, dsc0, g0, s0, is0, gs0, ss0,
                  si1, g1, gs1, is1)
        _halfstep(b0 + 1, si1, di1, wv1, dsc1, g1, s1, is1, gs1, ss1,
                  si0, g0, gs0, is0)
        return carry
    lax.fori_loop(0, NBLK // 2, _pair, 0)

    pltpu.make_async_copy(s0, acc.at[dsc0], ss0).wait()
    pltpu.make_async_copy(s1, acc.at[dsc1], ss1).wait()

    plsc.subcore_barrier()

    obase = c * N + rbase

    def _wout(k, carry):
        pltpu.sync_copy(acc.at[pl.ds(rbase + k * B, B)],
                        out_hbm.at[pl.ds(obase + k * B, B)])
        return carry
    lax.fori_loop(0, nfull, _wout, 0)

    @pl.when(rem > 0)
    def _():
        pltpu.sync_copy(acc.at[pl.ds(rbase + nfull * B, 64)],
                        out_hbm.at[pl.ds(obase + nfull * B, 64)])


@functools.cache
def _sc_agg():
    mesh = plsc.VectorSubcoreMesh(
        core_axis_name="c", subcore_axis_name="s",
        num_cores=NC, num_subcores=NS)
    return pl.kernel(
        _sc_agg_body,
        out_type=jax.ShapeDtypeStruct((NC * N, W), F32),
        mesh=mesh,
        compiler_params=pltpu.CompilerParams(
            needs_layout_passes=False, use_tc_tiling_on_sc=False),
        scratch_types=[
            pltpu.VMEM((B,), jnp.int32),
            pltpu.VMEM((B,), jnp.int32),
            pltpu.VMEM((B,), jnp.int32),
            pltpu.VMEM((B,), jnp.int32),
            pltpu.VMEM((B,), F32),
            pltpu.VMEM((B,), F32),
            pltpu.VMEM((B,), jnp.int32),
            pltpu.VMEM((B,), jnp.int32),
            pltpu.VMEM((B, W), F32),
            pltpu.VMEM((B, W), F32),
            pltpu.VMEM((B, W), F32),
            pltpu.VMEM((B, W), F32),
            pltpu.VMEM_SHARED((N, W), F32),
            pltpu.SemaphoreType.DMA,
            pltpu.SemaphoreType.DMA,
            pltpu.SemaphoreType.DMA,
            pltpu.SemaphoreType.DMA,
            pltpu.SemaphoreType.DMA,
            pltpu.SemaphoreType.DMA,
        ],
    )


def kernel(features, edge_index, W_in, b_in, W_g, a_src, a_dst, b_g,
           W_out, b_out):
    src2 = edge_index[0].astype(jnp.int32).reshape(E // B, B)
    dst2 = edge_index[1].astype(jnp.int32).reshape(E // B, B)
    src3 = src2.reshape(TILES, BPT_W, B)
    dst3 = dst2.reshape(TILES, BPT_W, B)
    b_in2 = b_in.reshape(1, H)
    b_g2 = b_g.reshape(1, H)
    b_out2 = b_out.reshape(1, H)
    a_s = a_src.reshape(H, 1)
    a_d = a_dst.reshape(H, 1)

    sc_w = _sc_w()
    sc_agg = _sc_agg()
    h0, whp0, es0, ed0 = _dense_in(features, W_in, b_in2, W_g, a_s, a_d)
    w0 = sc_w(src3, dst3, es0.reshape(N), ed0.reshape(N))
    acc0 = sc_agg(src2, dst2, w0.reshape(E // B, B),
                  whp0.reshape(NB * N, W))
    h1, whp1, es1, ed1 = _dense_mid(acc0.reshape(NC, N, W), W_g, b_g2,
                                    a_s, a_d)
    w1 = sc_w(src3, dst3, es1.reshape(N), ed1.reshape(N))
    acc1 = sc_agg(src2, dst2, w1.reshape(E // B, B),
                  whp1.reshape(NB * N, W))
    return _dense_out(acc1.reshape(NC, N, W), b_g2, h0, h1, W_out, b_out2)
